# baseline (speedup 1.0000x reference)
import functools

import jax
import jax.numpy as jnp
from jax.experimental import pallas as pl
from jax.experimental.pallas import tpu as pltpu

N = 10000
E = 160000
D = 256
H = 4
DH = D // H

_MBLK = 1000


def _qkv_body(emb_ref, w_ref, out_ref):
    out_ref[...] = jnp.dot(emb_ref[...], w_ref[...],
                           preferred_element_type=jnp.float32)


def _qkv_matmul(embeds, w):
    grid = (N // _MBLK,)
    return pl.pallas_call(
        _qkv_body,
        grid=grid,
        in_specs=[
            pl.BlockSpec((_MBLK, D), lambda i: (i, 0)),
            pl.BlockSpec((D, 3 * D), lambda i: (0, 0)),
        ],
        out_specs=pl.BlockSpec((_MBLK, 3 * D), lambda i: (i, 0)),
        out_shape=jax.ShapeDtypeStruct((N, 3 * D), jnp.float32),
    )(embeds, w)


def kernel(adj, embeds, qTrans, kTrans, vTrans):
    rows = adj[0, :]
    cols = adj[1, :]
    w = jnp.concatenate([qTrans, kTrans, vTrans], axis=1)
    qkv = _qkv_matmul(embeds, w)
    q = qkv[:, :D]
    k = qkv[:, D:2 * D]
    v = qkv[:, 2 * D:]

    qe = jnp.take(q, rows, axis=0).reshape(-1, H, DH)
    ke = jnp.take(k, cols, axis=0).reshape(-1, H, DH)
    ve = jnp.take(v, cols, axis=0).reshape(-1, H, DH)
    att = jnp.einsum('ehd,ehd->eh', qe, ke)
    att = jnp.clip(att, -10.0, 10.0)
    expAtt = jnp.exp(att)
    attNorm = jnp.zeros((N, H), dtype=jnp.float32).at[rows].add(expAtt)
    attNorm = jnp.take(attNorm, rows, axis=0)
    att = expAtt / (attNorm + 1e-08)
    resEmbeds = jnp.einsum('eh,ehd->ehd', att, ve).reshape(-1, D)
    out = jnp.zeros((N, D), dtype=jnp.float32).at[rows].add(resEmbeds)
    return (out, att)
